# Initial kernel scaffold; baseline (speedup 1.0000x reference)
import jax
import jax.numpy as jnp
from jax.experimental import pallas as pl


def kernel(x, edge_index, batch, edge_weight, W1_rel, b1_rel, W1_root, W2_rel, b2_rel, W2_root, W3_rel, b3_rel, W3_root, W_lin, b_lin):
    raise NotImplementedError("write your pallas kernel here")


# speedup vs baseline: 4.7091x; 4.7091x over previous
import functools

import jax
import jax.numpy as jnp
from jax import lax
from jax.experimental import pallas as pl
from jax.experimental.pallas import tpu as pltpu
from jax.experimental.pallas import tpu_sc as plsc

N = 10000
E = 320000
H = 128
G = 64
C = 16

CHUNK = 128
NTILES = 32
NCHUNKS = E // CHUNK
STRIPE = 624
REM = N - 16 * STRIPE
ZROWS = 208


def _sc_aggregate(x, src, dst, ew):
    mesh = plsc.VectorSubcoreMesh(core_axis_name="c", subcore_axis_name="s")

    @functools.partial(
        pl.kernel,
        mesh=mesh,
        out_type=jax.ShapeDtypeStruct((2, N, H), jnp.float32),
        scratch_types=[
            pltpu.VMEM_SHARED((N, H), jnp.float32),
            pltpu.VMEM((CHUNK,), jnp.int32),
            pltpu.VMEM((CHUNK,), jnp.int32),
            pltpu.VMEM((CHUNK,), jnp.float32),
            pltpu.VMEM((CHUNK, H), jnp.float32),
            pltpu.VMEM((ZROWS, H), jnp.float32),
            pltpu.SemaphoreType.DMA,
        ],
    )
    def agg_kernel(x_hbm, src_hbm, dst_hbm, ew_hbm, out_hbm,
                   acc, src_v, dst_v, ew_v, rows_v, zbuf, sem):
        cid = lax.axis_index("c")
        sid = lax.axis_index("s")
        wid = sid * 2 + cid

        def zrow(r, carry):
            for j in range(H // 16):
                zbuf[r, pl.ds(j * 16, 16)] = jnp.zeros((16,), jnp.float32)
            return carry
        lax.fori_loop(0, ZROWS, zrow, 0)
        for t in range(STRIPE // ZROWS):
            pltpu.sync_copy(zbuf, acc.at[pl.ds(sid * STRIPE + t * ZROWS, ZROWS)])

        @pl.when(sid == 0)
        def _zero_rem():
            pltpu.sync_copy(zbuf.at[pl.ds(0, REM)], acc.at[pl.ds(16 * STRIPE, REM)])
        plsc.subcore_barrier()

        nch = (NCHUNKS // NTILES) + jnp.where(wid < NCHUNKS % NTILES, 1, 0)

        def chunk_body(i, carry):
            base = (wid + NTILES * i) * CHUNK
            pltpu.sync_copy(src_hbm.at[pl.ds(base, CHUNK)], src_v)
            pltpu.sync_copy(dst_hbm.at[pl.ds(base, CHUNK)], dst_v)
            pltpu.sync_copy(ew_hbm.at[pl.ds(base, CHUNK)], ew_v)
            pltpu.async_copy(x_hbm.at[src_v], rows_v, sem).wait()

            def edge_body(g, carry2):
                wv = ew_v[pl.ds(g * 16, 16)]
                for u in range(16):
                    w = wv[u]
                    k = g * 16 + u
                    for j in range(H // 16):
                        rows_v[k, pl.ds(j * 16, 16)] = rows_v[k, pl.ds(j * 16, 16)] * w
                return carry2
            lax.fori_loop(0, CHUNK // 16, edge_body, 0)

            pltpu.sync_copy(rows_v, acc.at[dst_v], add=True)
            return carry
        lax.fori_loop(0, nch, chunk_body, 0)

        plsc.subcore_barrier()
        pltpu.sync_copy(
            acc.at[pl.ds(sid * STRIPE, STRIPE)],
            out_hbm.at[cid, pl.ds(sid * STRIPE, STRIPE)],
        )

        @pl.when(sid == 0)
        def _write_rem():
            pltpu.sync_copy(
                acc.at[pl.ds(16 * STRIPE, REM)],
                out_hbm.at[cid, pl.ds(16 * STRIPE, REM)],
            )

    return agg_kernel(x, src, dst, ew)


BN = 1000


def _tc_layer(parts, xin, w_rel, w_root, b, relu):
    def body(p_ref, x_ref, wr_ref, wt_ref, b_ref, o_ref):
        agg = p_ref[0] + p_ref[1]
        h = lax.dot_general(agg, wr_ref[...], (((1,), (1,)), ((), ())),
                            preferred_element_type=jnp.float32)
        h = h + lax.dot_general(x_ref[...], wt_ref[...], (((1,), (1,)), ((), ())),
                                preferred_element_type=jnp.float32)
        h = h + b_ref[...]
        o_ref[...] = jnp.maximum(h, 0.0) if relu else h

    return pl.pallas_call(
        body,
        grid=(N // BN,),
        in_specs=[
            pl.BlockSpec((2, BN, H), lambda i: (0, i, 0)),
            pl.BlockSpec((BN, H), lambda i: (i, 0)),
            pl.BlockSpec((H, H), lambda i: (0, 0)),
            pl.BlockSpec((H, H), lambda i: (0, 0)),
            pl.BlockSpec((1, H), lambda i: (0, 0)),
        ],
        out_specs=pl.BlockSpec((BN, H), lambda i: (i, 0)),
        out_shape=jax.ShapeDtypeStruct((N, H), jnp.float32),
    )(parts, xin, w_rel, w_root, b)


def _tc_final(parts, xin, w_rel, w_root, b, batch2d, w_lin, b_lin):
    nsteps = N // BN

    def body(p_ref, x_ref, wr_ref, wt_ref, b_ref, bt_ref, wl_ref, bl_ref,
             pooled_ref, out_ref, sums, cnts):
        i = pl.program_id(0)
        agg = p_ref[0] + p_ref[1]
        h = lax.dot_general(agg, wr_ref[...], (((1,), (1,)), ((), ())),
                            preferred_element_type=jnp.float32)
        h = h + lax.dot_general(x_ref[...], wt_ref[...], (((1,), (1,)), ((), ())),
                                preferred_element_type=jnp.float32)
        h = h + b_ref[...]

        onehot = (bt_ref[...] == lax.broadcasted_iota(jnp.int32, (BN, G), 1))
        onehot = onehot.astype(jnp.float32)

        @pl.when(i == 0)
        def _init():
            sums[...] = jnp.zeros_like(sums)
            cnts[...] = jnp.zeros_like(cnts)

        sums[...] += lax.dot_general(onehot, h, (((0,), (0,)), ((), ())),
                                     preferred_element_type=jnp.float32)
        cnts[...] += lax.dot_general(onehot, jnp.ones_like(h),
                                     (((0,), (0,)), ((), ())),
                                     preferred_element_type=jnp.float32)

        @pl.when(i == nsteps - 1)
        def _fin():
            pooled = sums[...] / jnp.maximum(cnts[...], 1.0)
            pooled_ref[...] = pooled
            out_ref[...] = lax.dot_general(pooled, wl_ref[...],
                                           (((1,), (1,)), ((), ())),
                                           preferred_element_type=jnp.float32) + bl_ref[...]

    return pl.pallas_call(
        body,
        grid=(nsteps,),
        in_specs=[
            pl.BlockSpec((2, BN, H), lambda i: (0, i, 0)),
            pl.BlockSpec((BN, H), lambda i: (i, 0)),
            pl.BlockSpec((H, H), lambda i: (0, 0)),
            pl.BlockSpec((H, H), lambda i: (0, 0)),
            pl.BlockSpec((1, H), lambda i: (0, 0)),
            pl.BlockSpec((BN, 1), lambda i: (i, 0)),
            pl.BlockSpec((C, H), lambda i: (0, 0)),
            pl.BlockSpec((1, C), lambda i: (0, 0)),
        ],
        out_specs=[
            pl.BlockSpec((G, H), lambda i: (0, 0)),
            pl.BlockSpec((G, C), lambda i: (0, 0)),
        ],
        out_shape=[
            jax.ShapeDtypeStruct((G, H), jnp.float32),
            jax.ShapeDtypeStruct((G, C), jnp.float32),
        ],
        scratch_shapes=[
            pltpu.VMEM((G, H), jnp.float32),
            pltpu.VMEM((G, H), jnp.float32),
        ],
        compiler_params=pltpu.CompilerParams(
            dimension_semantics=("arbitrary",)),
    )(parts, xin, w_rel, w_root, b, batch2d, w_lin, b_lin)


def kernel(x, edge_index, batch, edge_weight, W1_rel, b1_rel, W1_root,
           W2_rel, b2_rel, W2_root, W3_rel, b3_rel, W3_root, W_lin, b_lin):
    src = edge_index[0]
    dst = edge_index[1]
    batch2d = batch.reshape(N, 1)

    parts = _sc_aggregate(x, src, dst, edge_weight)
    h1 = _tc_layer(parts, x, W1_rel, W1_root, b1_rel.reshape(1, H), relu=True)
    parts = _sc_aggregate(h1, src, dst, edge_weight)
    h2 = _tc_layer(parts, h1, W2_rel, W2_root, b2_rel.reshape(1, H), relu=True)
    parts = _sc_aggregate(h2, src, dst, edge_weight)
    pooled, out = _tc_final(parts, h2, W3_rel, W3_root, b3_rel.reshape(1, H),
                            batch2d, W_lin, b_lin.reshape(1, C))
    return (pooled, out)
